# baseline (speedup 1.0000x reference)
import functools

import jax
import jax.numpy as jnp
from jax import lax
from jax.experimental import pallas as pl
from jax.experimental.pallas import tpu as pltpu
from jax.experimental.pallas import tpu_sc as plsc

NUM_CORES = 2
NUM_SUBCORES = 16
NUM_WORKERS = NUM_CORES * NUM_SUBCORES
B = 16384
D = 64
VOCAB = 1000000
LANES = 16

CPAD = 1 << 20
HALF = CPAD // 2
WZ = HALF // NUM_SUBCORES
ZCH = 16384
CHUNK = 128
ROWS_PER_SUBCORE = B // NUM_SUBCORES
NCH = ROWS_PER_SUBCORE // CHUNK

BLK = 98304
NBF = VOCAB // BLK
GRID = NBF + 1


def _sc_counts(idx):
  mesh = plsc.VectorSubcoreMesh(
      core_axis_name="c", subcore_axis_name="s",
      num_cores=NUM_CORES, num_subcores=NUM_SUBCORES)

  @functools.partial(
      pl.kernel,
      out_type=jax.ShapeDtypeStruct((1, CPAD), jnp.float32),
      mesh=mesh,
      compiler_params=pltpu.CompilerParams(use_tc_tiling_on_sc=True),
      scratch_types=[
          pltpu.VMEM((NCH, CHUNK), jnp.int32),
          pltpu.VMEM((ZCH,), jnp.float32),
          pltpu.VMEM((CHUNK,), jnp.float32),
          pltpu.VMEM_SHARED((HALF + CHUNK,), jnp.float32),
          pltpu.SemaphoreType.DMA,
      ],
  )
  def body(idx_hbm, out_hbm, idx_v, zero_v, ones_v, cnt_sh, sem):
    cid = lax.axis_index("c")
    sid = lax.axis_index("s")

    descs = [
        pltpu.async_copy(
            idx_hbm.at[cid,
                       pl.ds(sid * ROWS_PER_SUBCORE + j * CHUNK, CHUNK)],
            idx_v.at[j], sem)
        for j in range(NCH)
    ]

    def zstore(i, _):
      zero_v[pl.ds(i * LANES, LANES)] = jnp.zeros((LANES,), jnp.float32)
      return 0
    lax.fori_loop(0, ZCH // LANES, zstore, 0)
    for j in range(CHUNK // LANES):
      ones_v[pl.ds(j * LANES, LANES)] = jnp.ones((LANES,), jnp.float32)

    for j in range(WZ // ZCH):
      pltpu.sync_copy(zero_v, cnt_sh.at[pl.ds(sid * WZ + j * ZCH, ZCH)])
    for d_ in descs:
      d_.wait()
    plsc.subcore_barrier()

    for j in range(NCH):
      pltpu.sync_copy(ones_v, cnt_sh.at[idx_v.at[j]], add=True)
    plsc.subcore_barrier()

    pltpu.sync_copy(cnt_sh.at[pl.ds(sid * WZ, WZ)],
                    out_hbm.at[0, pl.ds(cid * HALF + sid * WZ, WZ)])

  return body(idx)


def _tc_scan(wt, counts):
  def body(wt_ref, c_ref, o_ref):
    q = pl.program_id(0)

    @pl.when(q == 0)
    def _():
      o_ref[...] = jnp.zeros((1, D), jnp.float32)

    cc = c_ref[0, :]
    cols = q * BLK + jax.lax.broadcasted_iota(jnp.int32, (1, BLK), 1)
    masked = jnp.where(cols < VOCAB, wt_ref[...] * cc[None, :], 0.0)
    o_ref[...] += jnp.sum(masked, axis=1).reshape(1, D)

    @pl.when(q == GRID - 1)
    def _():
      o_ref[...] *= 1.0 / B

  return pl.pallas_call(
      body,
      grid=(GRID,),
      in_specs=[pl.BlockSpec((D, BLK), lambda q: (0, q)),
                pl.BlockSpec((1, BLK), lambda q: (0, q))],
      out_specs=pl.BlockSpec((1, D), lambda q: (0, 0)),
      out_shape=jax.ShapeDtypeStruct((1, D), jnp.float32),
  )(wt, counts)


def kernel(input, weight):
  idx = input.astype(jnp.int32)
  dump = HALF + (jnp.arange(B, dtype=jnp.int32) % CHUNK)
  idx_l = jnp.stack([
      jnp.where(idx < HALF, idx, dump),
      jnp.where(idx >= HALF, idx - HALF, dump),
  ])
  counts = _sc_counts(idx_l)
  out = _tc_scan(weight.T, counts)
  return out.reshape(1, 1, D)


# speedup vs baseline: 1.0179x; 1.0179x over previous
import functools

import jax
import jax.numpy as jnp
from jax import lax
from jax.experimental import pallas as pl
from jax.experimental.pallas import tpu as pltpu
from jax.experimental.pallas import tpu_sc as plsc

NUM_CORES = 2
NUM_SUBCORES = 16
NUM_WORKERS = NUM_CORES * NUM_SUBCORES
B = 16384
D = 64
VOCAB = 1000000
LANES = 16

CPAD = 1 << 20
HALF = CPAD // 2
WZ = HALF // NUM_SUBCORES
ZCH = 16384
CHUNK = 128
ROWS_PER_SUBCORE = B // NUM_SUBCORES
NCH = ROWS_PER_SUBCORE // CHUNK

BLK = 65536
NBF = VOCAB // BLK
GRID = NBF + 1


def _sc_counts(idx):
  mesh = plsc.VectorSubcoreMesh(
      core_axis_name="c", subcore_axis_name="s",
      num_cores=NUM_CORES, num_subcores=NUM_SUBCORES)

  @functools.partial(
      pl.kernel,
      out_type=jax.ShapeDtypeStruct((1, CPAD), jnp.float32),
      mesh=mesh,
      compiler_params=pltpu.CompilerParams(use_tc_tiling_on_sc=True),
      scratch_types=[
          pltpu.VMEM((NCH, CHUNK), jnp.int32),
          pltpu.VMEM((ZCH,), jnp.float32),
          pltpu.VMEM((CHUNK,), jnp.float32),
          pltpu.VMEM_SHARED((HALF + CHUNK,), jnp.float32),
          pltpu.SemaphoreType.DMA,
      ],
  )
  def body(idx_hbm, out_hbm, idx_v, zero_v, ones_v, cnt_sh, sem):
    cid = lax.axis_index("c")
    sid = lax.axis_index("s")

    descs = [
        pltpu.async_copy(
            idx_hbm.at[cid,
                       pl.ds(sid * ROWS_PER_SUBCORE + j * CHUNK, CHUNK)],
            idx_v.at[j], sem)
        for j in range(NCH)
    ]

    def zstore(i, _):
      zero_v[pl.ds(i * LANES, LANES)] = jnp.zeros((LANES,), jnp.float32)
      return 0
    lax.fori_loop(0, ZCH // LANES, zstore, 0)
    for j in range(CHUNK // LANES):
      ones_v[pl.ds(j * LANES, LANES)] = jnp.ones((LANES,), jnp.float32)

    for j in range(WZ // ZCH):
      pltpu.sync_copy(zero_v, cnt_sh.at[pl.ds(sid * WZ + j * ZCH, ZCH)])
    for d_ in descs:
      d_.wait()
    plsc.subcore_barrier()

    for j in range(NCH):
      pltpu.sync_copy(ones_v, cnt_sh.at[idx_v.at[j]], add=True)
    plsc.subcore_barrier()

    pltpu.sync_copy(cnt_sh.at[pl.ds(sid * WZ, WZ)],
                    out_hbm.at[0, pl.ds(cid * HALF + sid * WZ, WZ)])

  return body(idx)


def _tc_scan(wt, counts):
  def body(wt_ref, c_ref, o_ref):
    q = pl.program_id(0)

    @pl.when(q == 0)
    def _():
      o_ref[...] = jnp.zeros((1, D), jnp.float32)

    cc = c_ref[0, :]
    cols = q * BLK + jax.lax.broadcasted_iota(jnp.int32, (1, BLK), 1)
    masked = jnp.where(cols < VOCAB, wt_ref[...] * cc[None, :], 0.0)
    o_ref[...] += jnp.sum(masked, axis=1).reshape(1, D)

    @pl.when(q == GRID - 1)
    def _():
      o_ref[...] *= 1.0 / B

  return pl.pallas_call(
      body,
      grid=(GRID,),
      in_specs=[pl.BlockSpec((D, BLK), lambda q: (0, q)),
                pl.BlockSpec((1, BLK), lambda q: (0, q))],
      out_specs=pl.BlockSpec((1, D), lambda q: (0, 0)),
      out_shape=jax.ShapeDtypeStruct((1, D), jnp.float32),
  )(wt, counts)


def kernel(input, weight):
  idx = input.astype(jnp.int32)
  dump = HALF + (jnp.arange(B, dtype=jnp.int32) % CHUNK)
  idx_l = jnp.stack([
      jnp.where(idx < HALF, idx, dump),
      jnp.where(idx >= HALF, idx - HALF, dump),
  ])
  counts = _sc_counts(idx_l)
  out = _tc_scan(weight.T, counts)
  return out.reshape(1, 1, D)
